# baseline (device time: 44032 ns/iter reference)
import os

import jax
import jax.numpy as jnp
from jax import lax
from jax.experimental import pallas as pl
from jax.experimental.pallas import tpu as pltpu

SKIP_COMM = os.environ.get("SKIP_COMM", "0") == "1"
SKIP_COMPUTE = os.environ.get("SKIP_COMPUTE", "0") == "1"

N_DEV = 4
SQ = 512
D = 1024
HEADS = 8
DH = 128
SCALE = 0.08838834764831843
NR = 4
RB = SQ // NR


def kernel(x, Wq, Wo, Wk, Wv):
    x2 = x.reshape(SQ, D)

    def body(x_ref, wq_hbm, wo_hbm, wk_hbm, wv_hbm, out_ref,
             attn_ref, wq_v, wk_v, wv_v, wo_v, stage1, stage2,
             w_sems, send_sems, recv_sems):
        my = lax.axis_index("i")
        left = (my + N_DEV - 1) % N_DEV
        right = (my + 1) % N_DEV

        if not SKIP_COMPUTE:
            cp_q = pltpu.make_async_copy(wq_hbm, wq_v, w_sems.at[0])
            cp_k = pltpu.make_async_copy(wk_hbm, wk_v, w_sems.at[1])
            cp_v = pltpu.make_async_copy(wv_hbm, wv_v, w_sems.at[2])
            cp_o = pltpu.make_async_copy(wo_hbm, wo_v, w_sems.at[3])
            cp_q.start()
            cp_k.start()
            cp_v.start()
            cp_o.start()

        barrier_sem = pltpu.get_barrier_semaphore()
        for nbr in (left, right):
            pl.semaphore_signal(
                barrier_sem, inc=1,
                device_id=(nbr,), device_id_type=pl.DeviceIdType.MESH,
            )
        pl.semaphore_wait(barrier_sem, 2)

        xbit = (my // 2) % 2
        ybit = ((my + 1) // 2) % 2
        yp = my ^ 1
        xp = 3 - my
        H = D // 2
        QT = H // 4

        a_half = ybit * (2 * QT)
        a_quar = a_half + xbit * QT
        b_half = H + xbit * (2 * QT)
        b_quar = b_half + ybit * QT

        def bspecs(s):
            if s == 0:
                return ((0, yp, (1 - ybit) * 2 * QT, 2 * QT, a_half, False),
                        (1, xp, H + (1 - xbit) * 2 * QT, 2 * QT, b_half, False))
            if s == 1:
                return ((0, xp, a_half + (1 - xbit) * QT, QT, a_quar, False),
                        (1, yp, b_half + (1 - ybit) * QT, QT, b_quar, False))
            if s == 2:
                return ((0, xp, a_quar, QT, None, True),
                        (1, yp, b_quar, QT, None, True))
            return ((0, yp, a_half, 2 * QT, None, True),
                    (1, xp, b_half, 2 * QT, None, True))

        inflight = {}

        def start_step(r, s):
            if SKIP_COMM:
                return
            rows = pl.ds(r * RB, RB)
            lst = []
            for d, peer, send_start, width, _, direct in bspecs(s):
                src = out_ref.at[rows, pl.ds(send_start, width)]
                if direct:
                    dst = src
                else:
                    dst = (stage1 if width == 2 * QT else stage2).at[d, r]
                rdma = pltpu.make_async_remote_copy(
                    src_ref=src,
                    dst_ref=dst,
                    send_sem=send_sems.at[d, s, r],
                    recv_sem=recv_sems.at[d, s, r],
                    device_id=(peer,),
                    device_id_type=pl.DeviceIdType.MESH,
                )
                rdma.start()
                lst.append(rdma)
            inflight[(r, s)] = lst

        def finish_step(r, s):
            if SKIP_COMM:
                return
            rows = pl.ds(r * RB, RB)
            for rdma in inflight.pop((r, s)):
                rdma.wait()
            for d, peer, send_start, width, recv_start, direct in bspecs(s):
                if not direct:
                    st = stage1 if width == 2 * QT else stage2
                    sl = pl.ds(recv_start, width)
                    out_ref[rows, sl] = out_ref[rows, sl] + st[d, r, :, :]

        if SKIP_COMPUTE:
            out_ref[:, :] = x_ref[:, :]
            return

        xv = x_ref[:, :].astype(jnp.bfloat16)
        cp_q.wait()
        q = jnp.dot(xv, wq_v[:, :].astype(jnp.bfloat16),
                    preferred_element_type=jnp.float32)
        cp_k.wait()
        k = jnp.dot(xv, wk_v[:, :].astype(jnp.bfloat16),
                    preferred_element_type=jnp.float32)
        cp_v.wait()
        v = jnp.dot(xv, wv_v[:, :].astype(jnp.bfloat16),
                    preferred_element_type=jnp.float32)

        for r in range(NR):
            r0, r1 = r * RB, (r + 1) * RB
            for h in range(HEADS):
                qh = q[r0:r1, h * DH:(h + 1) * DH].astype(jnp.bfloat16)
                kh = k[:, h * DH:(h + 1) * DH].astype(jnp.bfloat16)
                vh = v[:, h * DH:(h + 1) * DH].astype(jnp.bfloat16)
                s = lax.dot_general(
                    qh, kh, (((1,), (1,)), ((), ())),
                    preferred_element_type=jnp.float32,
                ) * SCALE
                p = jnp.exp(s)
                denom = jnp.sum(p, axis=1, keepdims=True)
                oh = jnp.dot(p.astype(jnp.bfloat16), vh,
                             preferred_element_type=jnp.float32) / denom
                attn_ref[r0:r1, h * DH:(h + 1) * DH] = oh
            if r == 0:
                cp_o.wait()
            out_ref[r0:r1, :] = jnp.dot(
                attn_ref[r0:r1, :].astype(jnp.bfloat16),
                wo_v[:, :].astype(jnp.bfloat16),
                preferred_element_type=jnp.float32)
            start_step(r, 0)

        for s in range(1, 4):
            for r in range(NR):
                finish_step(r, s - 1)
                start_step(r, s)
        for r in range(NR):
            finish_step(r, 3)

    out = pl.pallas_call(
        body,
        out_shape=jax.ShapeDtypeStruct((SQ, D), jnp.float32),
        in_specs=[pl.BlockSpec(memory_space=pltpu.VMEM)]
        + [pl.BlockSpec(memory_space=pl.ANY)] * 4,
        out_specs=pl.BlockSpec(memory_space=pltpu.VMEM),
        scratch_shapes=[
            pltpu.VMEM((SQ, D), jnp.float32),
            pltpu.VMEM((D, D), jnp.float32),
            pltpu.VMEM((D, D), jnp.float32),
            pltpu.VMEM((D, D), jnp.float32),
            pltpu.VMEM((D, D), jnp.float32),
            pltpu.VMEM((2, NR, RB, D // 4), jnp.float32),
            pltpu.VMEM((2, NR, RB, D // 8), jnp.float32),
            pltpu.SemaphoreType.DMA((4,)),
            pltpu.SemaphoreType.DMA((2, 4, NR)),
            pltpu.SemaphoreType.DMA((2, 4, NR)),
        ],
        compiler_params=pltpu.CompilerParams(collective_id=0),
    )(x2, Wq, Wo, Wk, Wv)
    return out.reshape(1, SQ, D)


# device time: 43995 ns/iter; 1.0008x vs baseline; 1.0008x over previous
import os

import jax
import jax.numpy as jnp
from jax import lax
from jax.experimental import pallas as pl
from jax.experimental.pallas import tpu as pltpu

SKIP_COMM = os.environ.get("SKIP_COMM", "0") == "1"
SKIP_COMPUTE = os.environ.get("SKIP_COMPUTE", "0") == "1"

N_DEV = 4
SQ = 512
D = 1024
HEADS = 8
DH = 128
SCALE = 0.08838834764831843
NR = 4
RB = SQ // NR


def kernel(x, Wq, Wo, Wk, Wv):
    x2 = x.reshape(SQ, D)

    def body(x_ref, wq_hbm, wo_hbm, wk_hbm, wv_hbm, out_ref,
             attn_ref, wq_v, wk_v, wv_v, wo_v, stage1, stage2,
             w_sems, send_sems, recv_sems):
        my = lax.axis_index("i")
        left = (my + N_DEV - 1) % N_DEV
        right = (my + 1) % N_DEV

        if not SKIP_COMPUTE:
            cp_q = pltpu.make_async_copy(wq_hbm, wq_v, w_sems.at[0])
            cp_k = pltpu.make_async_copy(wk_hbm, wk_v, w_sems.at[1])
            cp_v = pltpu.make_async_copy(wv_hbm, wv_v, w_sems.at[2])
            cp_o = pltpu.make_async_copy(wo_hbm, wo_v, w_sems.at[3])
            cp_q.start()

        barrier_sem = pltpu.get_barrier_semaphore()
        for nbr in (left, right):
            pl.semaphore_signal(
                barrier_sem, inc=1,
                device_id=(nbr,), device_id_type=pl.DeviceIdType.MESH,
            )
        pl.semaphore_wait(barrier_sem, 2)

        xbit = (my // 2) % 2
        ybit = ((my + 1) // 2) % 2
        yp = my ^ 1
        xp = 3 - my
        H = D // 2
        QT = H // 4

        a_half = ybit * (2 * QT)
        a_quar = a_half + xbit * QT
        b_half = H + xbit * (2 * QT)
        b_quar = b_half + ybit * QT

        def bspecs(s):
            if s == 0:
                return ((0, yp, (1 - ybit) * 2 * QT, 2 * QT, a_half, False),
                        (1, xp, H + (1 - xbit) * 2 * QT, 2 * QT, b_half, False))
            if s == 1:
                return ((0, xp, a_half + (1 - xbit) * QT, QT, a_quar, False),
                        (1, yp, b_half + (1 - ybit) * QT, QT, b_quar, False))
            if s == 2:
                return ((0, xp, a_quar, QT, None, True),
                        (1, yp, b_quar, QT, None, True))
            return ((0, yp, a_half, 2 * QT, None, True),
                    (1, xp, b_half, 2 * QT, None, True))

        inflight = {}

        def start_step(r, s):
            if SKIP_COMM:
                return
            rows = pl.ds(r * RB, RB)
            lst = []
            for d, peer, send_start, width, _, direct in bspecs(s):
                src = out_ref.at[rows, pl.ds(send_start, width)]
                if direct:
                    dst = src
                else:
                    dst = (stage1 if width == 2 * QT else stage2).at[d, r]
                rdma = pltpu.make_async_remote_copy(
                    src_ref=src,
                    dst_ref=dst,
                    send_sem=send_sems.at[d, s, r],
                    recv_sem=recv_sems.at[d, s, r],
                    device_id=(peer,),
                    device_id_type=pl.DeviceIdType.MESH,
                )
                rdma.start()
                lst.append(rdma)
            inflight[(r, s)] = lst

        def finish_step(r, s):
            if SKIP_COMM:
                return
            rows = pl.ds(r * RB, RB)
            for rdma in inflight.pop((r, s)):
                rdma.wait()
            for d, peer, send_start, width, recv_start, direct in bspecs(s):
                if not direct:
                    st = stage1 if width == 2 * QT else stage2
                    sl = pl.ds(recv_start, width)
                    out_ref[rows, sl] = out_ref[rows, sl] + st[d, r, :, :]

        if SKIP_COMPUTE:
            out_ref[:, :] = x_ref[:, :]
            return

        xv = x_ref[:, :].astype(jnp.bfloat16)
        cp_q.wait()
        cp_k.start()
        q = jnp.dot(xv, wq_v[:, :].astype(jnp.bfloat16),
                    preferred_element_type=jnp.float32)
        cp_k.wait()
        cp_v.start()
        k = jnp.dot(xv, wk_v[:, :].astype(jnp.bfloat16),
                    preferred_element_type=jnp.float32)
        cp_v.wait()
        cp_o.start()
        v = jnp.dot(xv, wv_v[:, :].astype(jnp.bfloat16),
                    preferred_element_type=jnp.float32)

        for r in range(NR):
            r0, r1 = r * RB, (r + 1) * RB
            for h in range(HEADS):
                qh = q[r0:r1, h * DH:(h + 1) * DH].astype(jnp.bfloat16)
                kh = k[:, h * DH:(h + 1) * DH].astype(jnp.bfloat16)
                vh = v[:, h * DH:(h + 1) * DH].astype(jnp.bfloat16)
                s = lax.dot_general(
                    qh, kh, (((1,), (1,)), ((), ())),
                    preferred_element_type=jnp.float32,
                ) * SCALE
                p = jnp.exp(s)
                denom = jnp.sum(p, axis=1, keepdims=True)
                oh = jnp.dot(p.astype(jnp.bfloat16), vh,
                             preferred_element_type=jnp.float32) / denom
                attn_ref[r0:r1, h * DH:(h + 1) * DH] = oh
            if r == 0:
                cp_o.wait()
            out_ref[r0:r1, :] = jnp.dot(
                attn_ref[r0:r1, :].astype(jnp.bfloat16),
                wo_v[:, :].astype(jnp.bfloat16),
                preferred_element_type=jnp.float32)
            start_step(r, 0)

        for s in range(1, 4):
            for r in range(NR):
                finish_step(r, s - 1)
                start_step(r, s)
        for r in range(NR):
            finish_step(r, 3)

    out = pl.pallas_call(
        body,
        out_shape=jax.ShapeDtypeStruct((SQ, D), jnp.float32),
        in_specs=[pl.BlockSpec(memory_space=pltpu.VMEM)]
        + [pl.BlockSpec(memory_space=pl.ANY)] * 4,
        out_specs=pl.BlockSpec(memory_space=pltpu.VMEM),
        scratch_shapes=[
            pltpu.VMEM((SQ, D), jnp.float32),
            pltpu.VMEM((D, D), jnp.float32),
            pltpu.VMEM((D, D), jnp.float32),
            pltpu.VMEM((D, D), jnp.float32),
            pltpu.VMEM((D, D), jnp.float32),
            pltpu.VMEM((2, NR, RB, D // 4), jnp.float32),
            pltpu.VMEM((2, NR, RB, D // 8), jnp.float32),
            pltpu.SemaphoreType.DMA((4,)),
            pltpu.SemaphoreType.DMA((2, 4, NR)),
            pltpu.SemaphoreType.DMA((2, 4, NR)),
        ],
        compiler_params=pltpu.CompilerParams(collective_id=0),
    )(x2, Wq, Wo, Wk, Wv)
    return out.reshape(1, SQ, D)


# device time: 36636 ns/iter; 1.2019x vs baseline; 1.2009x over previous
import os

import jax
import jax.numpy as jnp
from jax import lax
from jax.experimental import pallas as pl
from jax.experimental.pallas import tpu as pltpu

SKIP_COMM = os.environ.get("SKIP_COMM", "0") == "1"
SKIP_COMPUTE = os.environ.get("SKIP_COMPUTE", "0") == "1"

N_DEV = 4
SQ = 512
D = 1024
HEADS = 8
DH = 128
SCALE = 0.08838834764831843
NR = 4
RB = SQ // NR


def kernel(x, Wq, Wo, Wk, Wv):
    x2 = x.reshape(SQ, D)

    def body(x_ref, wq_ref, wo_ref, wk_ref, wv_ref, out_ref,
             attn_ref, red_ref, stage1, stage2, send_sems, recv_sems):
        my = lax.axis_index("i")
        left = (my + N_DEV - 1) % N_DEV
        right = (my + 1) % N_DEV

        barrier_sem = pltpu.get_barrier_semaphore()
        for nbr in (left, right):
            pl.semaphore_signal(
                barrier_sem, inc=1,
                device_id=(nbr,), device_id_type=pl.DeviceIdType.MESH,
            )
        pl.semaphore_wait(barrier_sem, 2)

        xbit = (my // 2) % 2
        ybit = ((my + 1) // 2) % 2
        yp = my ^ 1
        xp = 3 - my
        H = D // 2
        QT = H // 4

        a_half = ybit * (2 * QT)
        a_quar = a_half + xbit * QT
        b_half = H + xbit * (2 * QT)
        b_quar = b_half + ybit * QT

        def bspecs(s):
            if s == 0:
                return ((0, yp, (1 - ybit) * 2 * QT, 2 * QT, a_half, False),
                        (1, xp, H + (1 - xbit) * 2 * QT, 2 * QT, b_half, False))
            if s == 1:
                return ((0, xp, a_half + (1 - xbit) * QT, QT, a_quar, False),
                        (1, yp, b_half + (1 - ybit) * QT, QT, b_quar, False))
            if s == 2:
                return ((0, xp, a_quar, QT, None, True),
                        (1, yp, b_quar, QT, None, True))
            return ((0, yp, a_half, 2 * QT, None, True),
                    (1, xp, b_half, 2 * QT, None, True))

        inflight = {}

        def start_step(r, s):
            if SKIP_COMM:
                return
            rows = pl.ds(r * RB, RB)
            lst = []
            for d, peer, send_start, width, _, direct in bspecs(s):
                src = red_ref.at[rows, pl.ds(send_start, width)]
                if direct:
                    dst = src
                else:
                    dst = (stage1 if width == 2 * QT else stage2).at[d, r]
                rdma = pltpu.make_async_remote_copy(
                    src_ref=src,
                    dst_ref=dst,
                    send_sem=send_sems.at[d, s, r],
                    recv_sem=recv_sems.at[d, s, r],
                    device_id=(peer,),
                    device_id_type=pl.DeviceIdType.MESH,
                )
                rdma.start()
                lst.append(rdma)
            inflight[(r, s)] = lst

        def finish_step(r, s):
            if SKIP_COMM:
                return
            rows = pl.ds(r * RB, RB)
            for rdma in inflight.pop((r, s)):
                rdma.wait()
            for d, peer, send_start, width, recv_start, direct in bspecs(s):
                if not direct:
                    st = stage1 if width == 2 * QT else stage2
                    sl = pl.ds(recv_start, width)
                    red_ref[rows, sl] = red_ref[rows, sl] + st[d, r, :, :]

        if SKIP_COMPUTE:
            out_ref[:, :] = x_ref[:, :]
            return

        xv = x_ref[:, :].astype(jnp.bfloat16)
        wq = wq_ref[:, :].astype(jnp.bfloat16)
        wk = wk_ref[:, :].astype(jnp.bfloat16)
        wv = wv_ref[:, :].astype(jnp.bfloat16)
        q = jnp.dot(xv, wq, preferred_element_type=jnp.float32)
        k = jnp.dot(xv, wk, preferred_element_type=jnp.float32)
        v = jnp.dot(xv, wv, preferred_element_type=jnp.float32)
        wo = wo_ref[:, :].astype(jnp.bfloat16)

        for r in range(NR):
            r0, r1 = r * RB, (r + 1) * RB
            for h in range(HEADS):
                qh = q[r0:r1, h * DH:(h + 1) * DH].astype(jnp.bfloat16)
                kh = k[:, h * DH:(h + 1) * DH].astype(jnp.bfloat16)
                vh = v[:, h * DH:(h + 1) * DH].astype(jnp.bfloat16)
                s = lax.dot_general(
                    qh, kh, (((1,), (1,)), ((), ())),
                    preferred_element_type=jnp.float32,
                ) * SCALE
                p = jnp.exp(s)
                denom = jnp.sum(p, axis=1, keepdims=True)
                oh = jnp.dot(p.astype(jnp.bfloat16), vh,
                             preferred_element_type=jnp.float32) / denom
                attn_ref[r0:r1, h * DH:(h + 1) * DH] = oh
            red_ref[r0:r1, :] = jnp.dot(
                attn_ref[r0:r1, :].astype(jnp.bfloat16), wo,
                preferred_element_type=jnp.float32).astype(jnp.bfloat16)
            start_step(r, 0)

        if SKIP_COMM:
            out_ref[:, :] = red_ref[:, :].astype(jnp.float32)
            return

        for s in range(1, 4):
            for r in range(NR):
                finish_step(r, s - 1)
                start_step(r, s)
        for r in range(NR):
            finish_step(r, 3)
            r0, r1 = r * RB, (r + 1) * RB
            out_ref[r0:r1, :] = red_ref[r0:r1, :].astype(jnp.float32)

    out = pl.pallas_call(
        body,
        out_shape=jax.ShapeDtypeStruct((SQ, D), jnp.float32),
        in_specs=[pl.BlockSpec(memory_space=pltpu.VMEM)] * 5,
        out_specs=pl.BlockSpec(memory_space=pltpu.VMEM),
        scratch_shapes=[
            pltpu.VMEM((SQ, D), jnp.float32),
            pltpu.VMEM((SQ, D), jnp.bfloat16),
            pltpu.VMEM((2, NR, RB, D // 4), jnp.bfloat16),
            pltpu.VMEM((2, NR, RB, D // 8), jnp.bfloat16),
            pltpu.SemaphoreType.DMA((2, 4, NR)),
            pltpu.SemaphoreType.DMA((2, 4, NR)),
        ],
        compiler_params=pltpu.CompilerParams(collective_id=0),
    )(x2, Wq, Wo, Wk, Wv)
    return out.reshape(1, SQ, D)


# device time: 35348 ns/iter; 1.2457x vs baseline; 1.0364x over previous
import os

import jax
import jax.numpy as jnp
from jax import lax
from jax.experimental import pallas as pl
from jax.experimental.pallas import tpu as pltpu

SKIP_COMM = os.environ.get("SKIP_COMM", "0") == "1"
SKIP_COMPUTE = os.environ.get("SKIP_COMPUTE", "0") == "1"

N_DEV = 4
SQ = 512
D = 1024
HEADS = 8
DH = 128
SCALE = 0.08838834764831843
NR = 2
RB = SQ // NR


def kernel(x, Wq, Wo, Wk, Wv):
    x2 = x.reshape(SQ, D).astype(jnp.bfloat16)
    Wq = Wq.astype(jnp.bfloat16)
    Wk = Wk.astype(jnp.bfloat16)
    Wv = Wv.astype(jnp.bfloat16)
    Wo = Wo.astype(jnp.bfloat16)

    def body(x_ref, wq_ref, wo_ref, wk_ref, wv_ref, out_ref,
             attn_ref, red_ref, stage1, stage2, send_sems, recv_sems):
        my = lax.axis_index("i")
        left = (my + N_DEV - 1) % N_DEV
        right = (my + 1) % N_DEV

        barrier_sem = pltpu.get_barrier_semaphore()
        for nbr in (left, right):
            pl.semaphore_signal(
                barrier_sem, inc=1,
                device_id=(nbr,), device_id_type=pl.DeviceIdType.MESH,
            )
        pl.semaphore_wait(barrier_sem, 2)

        xbit = (my // 2) % 2
        ybit = ((my + 1) // 2) % 2
        yp = my ^ 1
        xp = 3 - my
        H = D // 2
        QT = H // 4

        a_half = ybit * (2 * QT)
        a_quar = a_half + xbit * QT
        b_half = H + xbit * (2 * QT)
        b_quar = b_half + ybit * QT

        def bspecs(s):
            if s == 0:
                return ((0, yp, (1 - ybit) * 2 * QT, 2 * QT, a_half, False),
                        (1, xp, H + (1 - xbit) * 2 * QT, 2 * QT, b_half, False))
            if s == 1:
                return ((0, xp, a_half + (1 - xbit) * QT, QT, a_quar, False),
                        (1, yp, b_half + (1 - ybit) * QT, QT, b_quar, False))
            if s == 2:
                return ((0, xp, a_quar, QT, None, True),
                        (1, yp, b_quar, QT, None, True))
            return ((0, yp, a_half, 2 * QT, None, True),
                    (1, xp, b_half, 2 * QT, None, True))

        inflight = {}
        pending_sends = []

        def start_step(r, s):
            if SKIP_COMM:
                return
            rows = pl.ds(r * RB, RB)
            lst = []
            for d, peer, send_start, width, _, direct in bspecs(s):
                src = red_ref.at[rows, pl.ds(send_start, width)]
                if direct:
                    dst = src
                else:
                    dst = (stage1 if width == 2 * QT else stage2).at[d, r]
                rdma = pltpu.make_async_remote_copy(
                    src_ref=src,
                    dst_ref=dst,
                    send_sem=send_sems.at[d, s, r],
                    recv_sem=recv_sems.at[d, s, r],
                    device_id=(peer,),
                    device_id_type=pl.DeviceIdType.MESH,
                )
                rdma.start()
                lst.append(rdma)
            inflight[(r, s)] = lst

        def finish_step(r, s):
            if SKIP_COMM:
                return
            rows = pl.ds(r * RB, RB)
            for rdma in inflight.pop((r, s)):
                rdma.wait_recv()
                pending_sends.append(rdma)
            for d, peer, send_start, width, recv_start, direct in bspecs(s):
                if not direct:
                    st = stage1 if width == 2 * QT else stage2
                    sl = pl.ds(recv_start, width)
                    red_ref[rows, sl] = red_ref[rows, sl] + st[d, r, :, :]

        if SKIP_COMPUTE:
            out_ref[:, :] = x_ref[:, :].astype(jnp.float32)
            return

        xv = x_ref[:, :]
        q = jnp.dot(xv, wq_ref[:, :], preferred_element_type=jnp.float32)
        k = jnp.dot(xv, wk_ref[:, :], preferred_element_type=jnp.float32)
        v = jnp.dot(xv, wv_ref[:, :], preferred_element_type=jnp.float32)
        wo = wo_ref[:, :]

        for r in range(NR):
            r0, r1 = r * RB, (r + 1) * RB
            for h in range(HEADS):
                qh = q[r0:r1, h * DH:(h + 1) * DH].astype(jnp.bfloat16)
                kh = k[:, h * DH:(h + 1) * DH].astype(jnp.bfloat16)
                vh = v[:, h * DH:(h + 1) * DH].astype(jnp.bfloat16)
                s = lax.dot_general(
                    qh, kh, (((1,), (1,)), ((), ())),
                    preferred_element_type=jnp.float32,
                ) * SCALE
                p = jnp.exp(s)
                denom = jnp.sum(p, axis=1, keepdims=True)
                oh = jnp.dot(p.astype(jnp.bfloat16), vh,
                             preferred_element_type=jnp.float32) / denom
                attn_ref[r0:r1, h * DH:(h + 1) * DH] = oh
            red_ref[r0:r1, :] = jnp.dot(
                attn_ref[r0:r1, :].astype(jnp.bfloat16), wo,
                preferred_element_type=jnp.float32).astype(jnp.bfloat16)
            start_step(r, 0)

        if SKIP_COMM:
            out_ref[:, :] = red_ref[:, :].astype(jnp.float32)
            return

        for s in range(1, 4):
            for r in range(NR):
                finish_step(r, s - 1)
                start_step(r, s)
        for r in range(NR):
            finish_step(r, 3)
            r0, r1 = r * RB, (r + 1) * RB
            out_ref[r0:r1, :] = red_ref[r0:r1, :].astype(jnp.float32)

        for rdma in pending_sends:
            rdma.wait_send()

    out = pl.pallas_call(
        body,
        out_shape=jax.ShapeDtypeStruct((SQ, D), jnp.float32),
        in_specs=[pl.BlockSpec(memory_space=pltpu.VMEM)] * 5,
        out_specs=pl.BlockSpec(memory_space=pltpu.VMEM),
        scratch_shapes=[
            pltpu.VMEM((SQ, D), jnp.float32),
            pltpu.VMEM((SQ, D), jnp.bfloat16),
            pltpu.VMEM((2, NR, RB, D // 4), jnp.bfloat16),
            pltpu.VMEM((2, NR, RB, D // 8), jnp.bfloat16),
            pltpu.SemaphoreType.DMA((2, 4, NR)),
            pltpu.SemaphoreType.DMA((2, 4, NR)),
        ],
        compiler_params=pltpu.CompilerParams(collective_id=0),
    )(x2, Wq, Wo, Wk, Wv)
    return out.reshape(1, SQ, D)
